# baseline (speedup 1.0000x reference)
import functools

import jax
import jax.numpy as jnp
from jax import lax
from jax.experimental import pallas as pl
from jax.experimental.pallas import tpu as pltpu
from jax.experimental.pallas import tpu_sc as plsc

_THR = 0.5
_N = 256 * 32 * 32
_NW = 32
_CHUNK = _N // _NW
_L = 16


def _gate_body(e_hbm, out_hbm, e_v, o_v):
    wid = lax.axis_index("s") * 2 + lax.axis_index("c")
    base = wid * _CHUNK
    pltpu.sync_copy(e_hbm.at[pl.ds(base, _CHUNK)], e_v)

    iota = lax.iota(jnp.int32, _L)
    one = jnp.full((_L,), 1, jnp.int32)
    zero = jnp.full((_L,), 0, jnp.int32)

    def step(i, carry):
        v = e_v[pl.ds(i * _L, _L)]
        m = v > _THR
        fi = jnp.where(m, one, zero)
        co = jnp.where(m, zero, one)
        idx = 2 * _L * i + 2 * iota
        plsc.store_scatter(o_v, [idx], co)
        plsc.store_scatter(o_v, [idx + 1], fi)
        return carry

    lax.fori_loop(0, _CHUNK // _L, step, 0)
    pltpu.sync_copy(o_v, out_hbm.at[pl.ds(2 * base, 2 * _CHUNK)])


_gate_sc = functools.partial(
    pl.kernel,
    out_type=jax.ShapeDtypeStruct((2 * _N,), jnp.int32),
    mesh=plsc.VectorSubcoreMesh(core_axis_name="c", subcore_axis_name="s"),
    scratch_types=[
        pltpu.VMEM((_CHUNK,), jnp.float32),
        pltpu.VMEM((2 * _CHUNK,), jnp.int32),
    ],
    compiler_params=pltpu.CompilerParams(needs_layout_passes=False),
)(_gate_body)


def kernel(entropy):
    flat = entropy.reshape(_N)
    out = _gate_sc(flat)
    return out.reshape(256, 32, 32, 2)


# speedup vs baseline: 1.0563x; 1.0563x over previous
import functools

import jax
import jax.numpy as jnp
from jax import lax
from jax.experimental import pallas as pl
from jax.experimental.pallas import tpu as pltpu

_N = 256 * 32 * 32
_R = 2048
_C = 128


def _gate_body(e_ref, o_ref):
    e = e_ref[...]
    fine = (e > 0.5).astype(jnp.float32)
    j = lax.broadcasted_iota(jnp.int32, (_C, 2 * _C), 0)
    k = lax.broadcasted_iota(jnp.int32, (_C, 2 * _C), 1)
    p = (j == k // 2).astype(jnp.float32)
    dup = jax.lax.dot_general(
        fine, p, (((1,), (0,)), ((), ())),
        preferred_element_type=jnp.float32)
    parity = lax.broadcasted_iota(jnp.int32, (_R, 2 * _C), 1) % 2
    gate = jnp.where(parity == 1, dup, 1.0 - dup)
    o_ref[...] = gate.astype(jnp.int32)


_gate_tc = pl.pallas_call(
    _gate_body,
    out_shape=jax.ShapeDtypeStruct((_R, 2 * _C), jnp.int32),
)


def kernel(entropy):
    flat = entropy.reshape(_R, _C)
    out = _gate_tc(flat)
    return out.reshape(256, 32, 32, 2)


# speedup vs baseline: 37.0771x; 35.1018x over previous
import jax
import jax.numpy as jnp
from jax.experimental import pallas as pl


def _gate_body(e_ref, o_ref):
    e = e_ref[...]
    fi = (e > 0.5).astype(jnp.int32)
    co = 1 - fi
    coL, coR = co[:, :128], co[:, 128:]
    fiL, fiR = fi[:, :128], fi[:, 128:]
    q = jnp.stack([coL, fiL, coR, fiR], axis=1)
    o_ref[...] = q.reshape(-1, 128)


_GRID = 8
_RB = 1024 // _GRID

_gate_tc = pl.pallas_call(
    _gate_body,
    grid=(_GRID,),
    in_specs=[pl.BlockSpec((_RB, 256), lambda i: (i, 0))],
    out_specs=pl.BlockSpec((4 * _RB, 128), lambda i: (i, 0)),
    out_shape=jax.ShapeDtypeStruct((4096, 128), jnp.int32),
)


def kernel(entropy):
    e2d = entropy.transpose(1, 2, 0).reshape(1024, 256)
    out2d = _gate_tc(e2d)
    o = out2d.reshape(32, 32, 2, 2, 128)
    o = o.transpose(2, 4, 0, 1, 3)
    return o.reshape(256, 32, 32, 2)
